# baseline (speedup 1.0000x reference)
import functools

import jax
import jax.numpy as jnp
import numpy as np
from jax.experimental import pallas as pl
from jax.sharding import Mesh, PartitionSpec as P

def _make_sharded(fn, mesh, in_specs, out_specs):
    if hasattr(jax, "shard_map"):
        return jax.shard_map(fn, mesh=mesh, in_specs=in_specs,
                             out_specs=out_specs, check_vma=False)
    from jax.experimental.shard_map import shard_map
    return shard_map(fn, mesh=mesh, in_specs=in_specs,
                     out_specs=out_specs, check_rep=False)


def _hnet_kernel(hs_ref, qwT_ref, kwT_ref, out_ref, *, L, D):
    hs = hs_ref[0]
    qwT = qwT_ref[...]
    kwT = kwT_ref[...]

    q = jnp.dot(hs, qwT, preferred_element_type=jnp.float32)
    k = jnp.dot(hs, kwT, preferred_element_type=jnp.float32)

    nq = jnp.maximum(jnp.sqrt(jnp.sum(q * q, axis=1, keepdims=True)), 1e-12)
    nk = jnp.maximum(jnp.sqrt(jnp.sum(k * k, axis=1, keepdims=True)), 1e-12)

    zrow = jnp.zeros((1, D), dtype=jnp.float32)
    k_next = jnp.concatenate([k[1:], zrow], axis=0)
    nk_next = jnp.concatenate([nk[1:], jnp.ones((1, 1), jnp.float32)], axis=0)

    dot_qk = jnp.sum(q * k_next, axis=1, keepdims=True)
    cos = dot_qk / (nq * nk_next)

    pm = jnp.clip((1.0 - cos) * 0.5, 0.0, 1.0)
    prob = jnp.concatenate([jnp.ones((1, 1), jnp.float32), pm[:L - 1]], axis=0)

    mask = prob > 0.5
    a_col = jnp.where(mask, 1.0 - prob, 1.0)
    c_col = jnp.where(mask, prob, 0.0)

    W = 128
    NB = D // W
    a = jnp.broadcast_to(a_col, (L, W))
    c = jnp.broadcast_to(c_col, (L, W))
    bs = [c * hs[:, j * W:(j + 1) * W] for j in range(NB)]

    s = 1
    while s < L:
        zpad = jnp.zeros((s, W), jnp.float32)
        a_sh = jnp.concatenate([jnp.ones((s, W), jnp.float32), a[:L - s]], axis=0)
        bs = [b + a * jnp.concatenate([zpad, b[:L - s]], axis=0) for b in bs]
        a = a * a_sh
        s *= 2

    for j in range(NB):
        out_ref[0, :, j * W:(j + 1) * W] = hs[:, j * W:(j + 1) * W] + bs[j]


def _run_block(hidden_states, qwT, kwT):
    B, L, D = hidden_states.shape
    return pl.pallas_call(
        functools.partial(_hnet_kernel, L=L, D=D),
        grid=(B,),
        in_specs=[
            pl.BlockSpec((1, L, D), lambda b: (b, 0, 0)),
            pl.BlockSpec((D, D), lambda b: (0, 0)),
            pl.BlockSpec((D, D), lambda b: (0, 0)),
        ],
        out_specs=pl.BlockSpec((1, L, D), lambda b: (b, 0, 0)),
        out_shape=jax.ShapeDtypeStruct((B, L, D), hidden_states.dtype),
    )(hidden_states, qwT, kwT)


def kernel(hidden_states, q_weight, k_weight):
    B = hidden_states.shape[0]
    qwT = q_weight.T
    kwT = k_weight.T
    devs = jax.devices()
    ndev = 1
    for cand in (4, 2):
        if len(devs) >= cand and B % cand == 0:
            ndev = cand
            break
    if ndev == 1:
        return _run_block(hidden_states, qwT, kwT)
    mesh = Mesh(np.array(devs[:ndev]), ("d",))
    fn = _make_sharded(_run_block, mesh, (P("d"), P(), P()), P("d"))
    return fn(hidden_states, qwT, kwT)


# speedup vs baseline: 10.8237x; 10.8237x over previous
import functools

import jax
import jax.numpy as jnp
from jax.experimental import pallas as pl

_T = 128


def _hnet_kernel(hs_ref, qwT_ref, kwT_ref, out_ref, *, L, D):
    T = _T
    G = L // T
    f32 = jnp.float32
    hs = hs_ref[0]
    qwT = qwT_ref[...]
    kwT = kwT_ref[...]

    q = jnp.dot(hs, qwT, preferred_element_type=f32)
    k = jnp.dot(hs, kwT, preferred_element_type=f32)

    k_next = jnp.concatenate([k[1:], jnp.zeros((1, D), f32)], axis=0)

    ones_red = jnp.ones((D, T), f32)
    nq2 = jnp.dot(q * q, ones_red, preferred_element_type=f32)
    nk2 = jnp.dot(k_next * k_next, ones_red, preferred_element_type=f32)
    dqk = jnp.dot(q * k_next, ones_red, preferred_element_type=f32)

    nq = jnp.maximum(jnp.sqrt(nq2), 1e-12)
    nk = jnp.maximum(jnp.sqrt(nk2), 1e-12)
    cos = dqk / (nq * nk)

    pm = jnp.clip((1.0 - cos) * 0.5, 0.0, 1.0)
    prob = jnp.concatenate([jnp.ones((1, T), f32), pm[:L - 1]], axis=0)

    mask = prob > 0.5
    a = jnp.where(mask, 1.0 - prob, 1.0)
    c = jnp.where(mask, prob, 0.0)

    row = jax.lax.broadcasted_iota(jnp.int32, (L, T), 0)
    col = jax.lax.broadcasted_iota(jnp.int32, (L, T), 1)
    tmod = jnp.bitwise_and(row, T - 1)
    Lm = jnp.where(tmod == col, 1.0, 0.0).astype(f32)
    s = 1
    while s < T:
        valid = tmod >= s
        am = jnp.where(valid, a, 0.0)
        a_sh = jnp.concatenate([jnp.ones((s, T), f32), a[:L - s]], axis=0)
        a_sh = jnp.where(valid, a_sh, 1.0)
        Lm_sh = jnp.concatenate([jnp.zeros((s, T), f32), Lm[:L - s]], axis=0)
        Lm = Lm + am * Lm_sh
        a = a * a_sh
        s *= 2

    b_full = jnp.concatenate(
        [c * hs[:, j * T:(j + 1) * T] for j in range(D // T)], axis=1)

    s_locs = [
        jnp.dot(Lm[g * T:(g + 1) * T], b_full[g * T:(g + 1) * T],
                preferred_element_type=f32)
        for g in range(G)
    ]

    Sb = jnp.concatenate([sl[T - 1:T] for sl in s_locs], axis=0)
    Aa = jnp.concatenate(
        [a[g * T + T - 1:g * T + T] for g in range(G)], axis=0)
    s = 1
    while s < G:
        Sb_sh = jnp.concatenate([jnp.zeros((s, D), f32), Sb[:G - s]], axis=0)
        Aa_sh = jnp.concatenate([jnp.ones((s, T), f32), Aa[:G - s]], axis=0)
        Sb = Sb + jnp.concatenate(
            [Aa * Sb_sh[:, j * T:(j + 1) * T] for j in range(D // T)], axis=1)
        Aa = Aa * Aa_sh
        s *= 2
    S_prev = jnp.concatenate([jnp.zeros((1, D), f32), Sb[:G - 1]], axis=0)

    for g in range(G):
        r0 = g * T
        carry = jnp.broadcast_to(S_prev[g:g + 1, :], (T, D))
        a_pre = a[r0:r0 + T]
        corr = jnp.concatenate(
            [a_pre * carry[:, j * T:(j + 1) * T] for j in range(D // T)], axis=1)
        out_ref[0, r0:r0 + T, :] = hs[r0:r0 + T] + s_locs[g] + corr


def _run_block(hidden_states, qwT, kwT):
    B, L, D = hidden_states.shape
    return pl.pallas_call(
        functools.partial(_hnet_kernel, L=L, D=D),
        grid=(B,),
        in_specs=[
            pl.BlockSpec((1, L, D), lambda b: (b, 0, 0)),
            pl.BlockSpec((D, D), lambda b: (0, 0)),
            pl.BlockSpec((D, D), lambda b: (0, 0)),
        ],
        out_specs=pl.BlockSpec((1, L, D), lambda b: (b, 0, 0)),
        out_shape=jax.ShapeDtypeStruct((B, L, D), hidden_states.dtype),
    )(hidden_states, qwT, kwT)


def kernel(hidden_states, q_weight, k_weight):
    qwT = q_weight.T
    kwT = k_weight.T
    return _run_block(hidden_states, qwT, kwT)


# speedup vs baseline: 11.0280x; 1.0189x over previous
import functools

import jax
import jax.numpy as jnp
from jax.experimental import pallas as pl

_T = 128


def _hnet_kernel(hs_ref, qwT_ref, kwT_ref, out_ref, *, L, D):
    T = _T
    G = L // T
    f32 = jnp.float32
    hs = hs_ref[0]
    qwT = qwT_ref[...]
    kwT = kwT_ref[...]

    q = jnp.dot(hs, qwT, preferred_element_type=f32)
    k = jnp.dot(hs, kwT, preferred_element_type=f32)

    k_next = jnp.concatenate([k[1:], jnp.zeros((1, D), f32)], axis=0)

    nq = jnp.maximum(jnp.sqrt(jnp.sum(q * q, axis=1, keepdims=True)), 1e-12)
    nk2c = jnp.sum(k_next * k_next, axis=1, keepdims=True)
    nk = jnp.maximum(jnp.sqrt(nk2c), 1e-12)
    dqk = jnp.sum(q * k_next, axis=1, keepdims=True)
    cos = dqk / (nq * nk)

    pm = jnp.clip((1.0 - cos) * 0.5, 0.0, 1.0)
    prob = jnp.concatenate([jnp.ones((1, 1), f32), pm[:L - 1]], axis=0)

    mask = prob > 0.5
    a_col = jnp.where(mask, 1.0 - prob, 1.0)
    c_col = jnp.where(mask, prob, 0.0)
    a = jnp.broadcast_to(a_col, (L, T))
    c = jnp.broadcast_to(c_col, (L, T))

    row = jax.lax.broadcasted_iota(jnp.int32, (L, T), 0)
    col = jax.lax.broadcasted_iota(jnp.int32, (L, T), 1)
    tmod = jnp.bitwise_and(row, T - 1)
    Lm = jnp.where(tmod == col, 1.0, 0.0).astype(f32)
    s = 1
    while s < T:
        valid = tmod >= s
        am = jnp.where(valid, a, 0.0)
        a_sh = jnp.concatenate([jnp.ones((s, T), f32), a[:L - s]], axis=0)
        a_sh = jnp.where(valid, a_sh, 1.0)
        Lm_sh = jnp.concatenate([jnp.zeros((s, T), f32), Lm[:L - s]], axis=0)
        Lm = Lm + am * Lm_sh
        a = a * a_sh
        s *= 2

    b_full = jnp.concatenate(
        [c * hs[:, j * T:(j + 1) * T] for j in range(D // T)], axis=1)

    s_locs = [
        jnp.dot(Lm[g * T:(g + 1) * T], b_full[g * T:(g + 1) * T],
                preferred_element_type=f32)
        for g in range(G)
    ]

    Sb = jnp.concatenate([sl[T - 1:T] for sl in s_locs], axis=0)
    Aa = jnp.concatenate(
        [a[g * T + T - 1:g * T + T] for g in range(G)], axis=0)
    s = 1
    while s < G:
        Sb_sh = jnp.concatenate([jnp.zeros((s, D), f32), Sb[:G - s]], axis=0)
        Aa_sh = jnp.concatenate([jnp.ones((s, T), f32), Aa[:G - s]], axis=0)
        Sb = Sb + jnp.concatenate(
            [Aa * Sb_sh[:, j * T:(j + 1) * T] for j in range(D // T)], axis=1)
        Aa = Aa * Aa_sh
        s *= 2
    S_prev = jnp.concatenate([jnp.zeros((1, D), f32), Sb[:G - 1]], axis=0)

    for g in range(G):
        r0 = g * T
        carry = jnp.broadcast_to(S_prev[g:g + 1, :], (T, D))
        a_pre = a[r0:r0 + T]
        corr = jnp.concatenate(
            [a_pre * carry[:, j * T:(j + 1) * T] for j in range(D // T)], axis=1)
        out_ref[0, r0:r0 + T, :] = hs[r0:r0 + T] + s_locs[g] + corr


def _run_block(hidden_states, qwT, kwT):
    B, L, D = hidden_states.shape
    return pl.pallas_call(
        functools.partial(_hnet_kernel, L=L, D=D),
        grid=(B,),
        in_specs=[
            pl.BlockSpec((1, L, D), lambda b: (b, 0, 0)),
            pl.BlockSpec((D, D), lambda b: (0, 0)),
            pl.BlockSpec((D, D), lambda b: (0, 0)),
        ],
        out_specs=pl.BlockSpec((1, L, D), lambda b: (b, 0, 0)),
        out_shape=jax.ShapeDtypeStruct((B, L, D), hidden_states.dtype),
    )(hidden_states, qwT, kwT)


def kernel(hidden_states, q_weight, k_weight):
    qwT = q_weight.T
    kwT = k_weight.T
    return _run_block(hidden_states, qwT, kwT)


# speedup vs baseline: 11.5667x; 1.0488x over previous
import functools

import jax
import jax.numpy as jnp
from jax.experimental import pallas as pl

_T = 128


def _hnet_kernel(hs_ref, wT_ref, out_ref, *, L, D):
    T = _T
    G = L // T
    f32 = jnp.float32
    hs = hs_ref[0]
    wT = wT_ref[...]

    qk = jnp.dot(hs, wT, preferred_element_type=f32)
    q = qk[:, :D]
    k = qk[:, D:]

    k_next = jnp.concatenate([k[1:], jnp.zeros((1, D), f32)], axis=0)

    nq = jnp.maximum(jnp.sqrt(jnp.sum(q * q, axis=1, keepdims=True)), 1e-12)
    nk2c = jnp.sum(k_next * k_next, axis=1, keepdims=True)
    nk = jnp.maximum(jnp.sqrt(nk2c), 1e-12)
    dqk = jnp.sum(q * k_next, axis=1, keepdims=True)
    cos = dqk / (nq * nk)

    pm = jnp.clip((1.0 - cos) * 0.5, 0.0, 1.0)
    prob = jnp.concatenate([jnp.ones((1, 1), f32), pm[:L - 1]], axis=0)

    mask = prob > 0.5
    a_col = jnp.where(mask, 1.0 - prob, 1.0)
    c_col = jnp.where(mask, prob, 0.0)
    a = jnp.broadcast_to(a_col, (L, T))
    c = jnp.broadcast_to(c_col, (L, T))

    t_io = jax.lax.broadcasted_iota(jnp.int32, (G, T, T), 1)
    j_io = jax.lax.broadcasted_iota(jnp.int32, (G, T, T), 2)
    Lm3 = jnp.where(t_io == j_io, 1.0, 0.0).astype(f32)
    a3 = a.reshape(G, T, T)
    s = 1
    while s < T:
        a_sh = jnp.concatenate([jnp.ones((G, s, T), f32), a3[:, :T - s]], axis=1)
        Lm_sh = jnp.concatenate([jnp.zeros((G, s, T), f32), Lm3[:, :T - s]], axis=1)
        Lm3 = Lm3 + a3 * Lm_sh
        a3 = a3 * a_sh
        s *= 2
    Lm = Lm3.reshape(L, T)
    a = a3.reshape(L, T)

    b_full = jnp.concatenate(
        [c * hs[:, j * T:(j + 1) * T] for j in range(D // T)], axis=1)

    s_locs = [
        jnp.dot(Lm[g * T:(g + 1) * T], b_full[g * T:(g + 1) * T],
                preferred_element_type=f32)
        for g in range(G)
    ]

    Sb = jnp.concatenate([sl[T - 1:T] for sl in s_locs], axis=0)
    Aa = jnp.concatenate(
        [a[g * T + T - 1:g * T + T] for g in range(G)], axis=0)
    s = 1
    while s < G:
        Sb_sh = jnp.concatenate([jnp.zeros((s, D), f32), Sb[:G - s]], axis=0)
        Aa_sh = jnp.concatenate([jnp.ones((s, T), f32), Aa[:G - s]], axis=0)
        Sb = Sb + jnp.concatenate(
            [Aa * Sb_sh[:, j * T:(j + 1) * T] for j in range(D // T)], axis=1)
        Aa = Aa * Aa_sh
        s *= 2
    S_prev = jnp.concatenate([jnp.zeros((1, D), f32), Sb[:G - 1]], axis=0)

    for g in range(G):
        r0 = g * T
        carry = jnp.broadcast_to(S_prev[g:g + 1, :], (T, D))
        a_pre = a[r0:r0 + T]
        corr = jnp.concatenate(
            [a_pre * carry[:, j * T:(j + 1) * T] for j in range(D // T)], axis=1)
        out_ref[0, r0:r0 + T, :] = hs[r0:r0 + T] + s_locs[g] + corr


def _run_block(hidden_states, wT):
    B, L, D = hidden_states.shape
    return pl.pallas_call(
        functools.partial(_hnet_kernel, L=L, D=D),
        grid=(B,),
        in_specs=[
            pl.BlockSpec((1, L, D), lambda b: (b, 0, 0)),
            pl.BlockSpec((D, 2 * D), lambda b: (0, 0)),
        ],
        out_specs=pl.BlockSpec((1, L, D), lambda b: (b, 0, 0)),
        out_shape=jax.ShapeDtypeStruct((B, L, D), hidden_states.dtype),
    )(hidden_states, wT)


def kernel(hidden_states, q_weight, k_weight):
    wT = jnp.concatenate([q_weight, k_weight], axis=0).T
    return _run_block(hidden_states, wT)


# speedup vs baseline: 11.6272x; 1.0052x over previous
import functools

import jax
import jax.numpy as jnp
from jax.experimental import pallas as pl

_T = 128


def _hnet_kernel(hs_ref, w_ref, out_ref, *, L, D):
    T = _T
    G = L // T
    f32 = jnp.float32
    hs = hs_ref[0]
    w = w_ref[...]

    qk = jax.lax.dot_general(hs, w, (((1,), (1,)), ((), ())),
                             preferred_element_type=f32)
    q = qk[:, :D]
    k = qk[:, D:]

    k_next = jnp.concatenate([k[1:], jnp.zeros((1, D), f32)], axis=0)

    nq = jnp.maximum(jnp.sqrt(jnp.sum(q * q, axis=1, keepdims=True)), 1e-12)
    nk2c = jnp.sum(k_next * k_next, axis=1, keepdims=True)
    nk = jnp.maximum(jnp.sqrt(nk2c), 1e-12)
    dqk = jnp.sum(q * k_next, axis=1, keepdims=True)
    cos = dqk / (nq * nk)

    pm = jnp.clip((1.0 - cos) * 0.5, 0.0, 1.0)
    prob = jnp.concatenate([jnp.ones((1, 1), f32), pm[:L - 1]], axis=0)

    mask = prob > 0.5
    a_col = jnp.where(mask, 1.0 - prob, 1.0)
    c_col = jnp.where(mask, prob, 0.0)
    a = jnp.broadcast_to(a_col, (L, T))
    c = jnp.broadcast_to(c_col, (L, T))

    t_io = jax.lax.broadcasted_iota(jnp.int32, (G, T, T), 1)
    j_io = jax.lax.broadcasted_iota(jnp.int32, (G, T, T), 2)
    Lm3 = jnp.where(t_io == j_io, 1.0, 0.0).astype(f32)
    a3 = a.reshape(G, T, T)
    s = 1
    while s < T:
        a_sh = jnp.concatenate([jnp.ones((G, s, T), f32), a3[:, :T - s]], axis=1)
        Lm_sh = jnp.concatenate([jnp.zeros((G, s, T), f32), Lm3[:, :T - s]], axis=1)
        Lm3 = Lm3 + a3 * Lm_sh
        a3 = a3 * a_sh
        s *= 2
    Lm = Lm3.reshape(L, T)
    a = a3.reshape(L, T)

    b_full = jnp.concatenate(
        [c * hs[:, j * T:(j + 1) * T] for j in range(D // T)], axis=1)

    s_locs = [
        jnp.dot(Lm[g * T:(g + 1) * T], b_full[g * T:(g + 1) * T],
                preferred_element_type=f32)
        for g in range(G)
    ]

    Sb = jnp.concatenate([sl[T - 1:T] for sl in s_locs], axis=0)
    Aa = jnp.concatenate(
        [a[g * T + T - 1:g * T + T] for g in range(G)], axis=0)
    s = 1
    while s < G:
        Sb_sh = jnp.concatenate([jnp.zeros((s, D), f32), Sb[:G - s]], axis=0)
        Aa_sh = jnp.concatenate([jnp.ones((s, T), f32), Aa[:G - s]], axis=0)
        Sb = Sb + jnp.concatenate(
            [Aa * Sb_sh[:, j * T:(j + 1) * T] for j in range(D // T)], axis=1)
        Aa = Aa * Aa_sh
        s *= 2
    S_prev = jnp.concatenate([jnp.zeros((1, D), f32), Sb[:G - 1]], axis=0)

    for g in range(G):
        r0 = g * T
        carry = jnp.broadcast_to(S_prev[g:g + 1, :], (T, D))
        a_pre = a[r0:r0 + T]
        corr = jnp.concatenate(
            [a_pre * carry[:, j * T:(j + 1) * T] for j in range(D // T)], axis=1)
        out_ref[0, r0:r0 + T, :] = hs[r0:r0 + T] + s_locs[g] + corr


def _run_block(hidden_states, w):
    B, L, D = hidden_states.shape
    return pl.pallas_call(
        functools.partial(_hnet_kernel, L=L, D=D),
        grid=(B,),
        in_specs=[
            pl.BlockSpec((1, L, D), lambda b: (b, 0, 0)),
            pl.BlockSpec((2 * D, D), lambda b: (0, 0)),
        ],
        out_specs=pl.BlockSpec((1, L, D), lambda b: (b, 0, 0)),
        out_shape=jax.ShapeDtypeStruct((B, L, D), hidden_states.dtype),
    )(hidden_states, w)


def kernel(hidden_states, q_weight, k_weight):
    w = jnp.concatenate([q_weight, k_weight], axis=0)
    return _run_block(hidden_states, w)


# speedup vs baseline: 12.6743x; 1.0901x over previous
import functools

import jax
import jax.numpy as jnp
from jax.experimental import pallas as pl

_T = 128


def _hnet_kernel(hs_ref, qw_ref, kw_ref, out_ref, *, L, D):
    T = _T
    G = L // T
    f32 = jnp.float32
    hs = hs_ref[0]
    qw = qw_ref[...]
    kw = kw_ref[...]

    dn = (((1,), (1,)), ((), ()))
    q = jax.lax.dot_general(hs, qw, dn, preferred_element_type=f32)
    k = jax.lax.dot_general(hs, kw, dn, preferred_element_type=f32)

    k_next = jnp.concatenate([k[1:], jnp.zeros((1, D), f32)], axis=0)

    nq = jnp.maximum(jnp.sqrt(jnp.sum(q * q, axis=1, keepdims=True)), 1e-12)
    nk2c = jnp.sum(k_next * k_next, axis=1, keepdims=True)
    nk = jnp.maximum(jnp.sqrt(nk2c), 1e-12)
    dqk = jnp.sum(q * k_next, axis=1, keepdims=True)
    cos = dqk / (nq * nk)

    pm = jnp.clip((1.0 - cos) * 0.5, 0.0, 1.0)
    prob = jnp.concatenate([jnp.ones((1, 1), f32), pm[:L - 1]], axis=0)

    mask = prob > 0.5
    a_col = jnp.where(mask, 1.0 - prob, 1.0)
    c_col = jnp.where(mask, prob, 0.0)
    a = jnp.broadcast_to(a_col, (L, T))
    c = jnp.broadcast_to(c_col, (L, T))

    t_io = jax.lax.broadcasted_iota(jnp.int32, (G, T, T), 1)
    j_io = jax.lax.broadcasted_iota(jnp.int32, (G, T, T), 2)
    Lm3 = jnp.where(t_io == j_io, 1.0, 0.0).astype(f32)
    a3 = a.reshape(G, T, T)
    s = 1
    while s < T:
        a_sh = jnp.concatenate([jnp.ones((G, s, T), f32), a3[:, :T - s]], axis=1)
        Lm_sh = jnp.concatenate([jnp.zeros((G, s, T), f32), Lm3[:, :T - s]], axis=1)
        Lm3 = Lm3 + a3 * Lm_sh
        a3 = a3 * a_sh
        s *= 2
    Lm = Lm3.reshape(L, T)
    a = a3.reshape(L, T)

    b_full = jnp.concatenate(
        [c * hs[:, j * T:(j + 1) * T] for j in range(D // T)], axis=1)

    s_locs = [
        jnp.dot(Lm[g * T:(g + 1) * T], b_full[g * T:(g + 1) * T],
                preferred_element_type=f32)
        for g in range(G)
    ]

    Sb = jnp.concatenate([sl[T - 1:T] for sl in s_locs], axis=0)
    Aa = jnp.concatenate(
        [a[g * T + T - 1:g * T + T] for g in range(G)], axis=0)
    s = 1
    while s < G:
        Sb_sh = jnp.concatenate([jnp.zeros((s, D), f32), Sb[:G - s]], axis=0)
        Aa_sh = jnp.concatenate([jnp.ones((s, T), f32), Aa[:G - s]], axis=0)
        Sb = Sb + jnp.concatenate(
            [Aa * Sb_sh[:, j * T:(j + 1) * T] for j in range(D // T)], axis=1)
        Aa = Aa * Aa_sh
        s *= 2
    S_prev = jnp.concatenate([jnp.zeros((1, D), f32), Sb[:G - 1]], axis=0)

    for g in range(G):
        r0 = g * T
        carry = jnp.broadcast_to(S_prev[g:g + 1, :], (T, D))
        a_pre = a[r0:r0 + T]
        corr = jnp.concatenate(
            [a_pre * carry[:, j * T:(j + 1) * T] for j in range(D // T)], axis=1)
        out_ref[0, r0:r0 + T, :] = hs[r0:r0 + T] + s_locs[g] + corr


def kernel(hidden_states, q_weight, k_weight):
    B, L, D = hidden_states.shape
    return pl.pallas_call(
        functools.partial(_hnet_kernel, L=L, D=D),
        grid=(B,),
        in_specs=[
            pl.BlockSpec((1, L, D), lambda b: (b, 0, 0)),
            pl.BlockSpec((D, D), lambda b: (0, 0)),
            pl.BlockSpec((D, D), lambda b: (0, 0)),
        ],
        out_specs=pl.BlockSpec((1, L, D), lambda b: (b, 0, 0)),
        out_shape=jax.ShapeDtypeStruct((B, L, D), hidden_states.dtype),
    )(hidden_states, q_weight, k_weight)


# speedup vs baseline: 12.7261x; 1.0041x over previous
import functools

import jax
import jax.numpy as jnp
from jax.experimental import pallas as pl

_T = 128


def _hnet_kernel(hs_ref, qw_ref, kw_ref, out_ref, *, L, D):
    T = _T
    G = L // T
    f32 = jnp.float32
    hs = hs_ref[0]
    qw = qw_ref[...]
    kw = kw_ref[...]

    dn = (((1,), (1,)), ((), ()))
    q = jax.lax.dot_general(hs, qw, dn, preferred_element_type=f32)
    k = jax.lax.dot_general(hs, kw, dn, preferred_element_type=f32)

    k_next = jnp.concatenate([k[1:], jnp.zeros((1, D), f32)], axis=0)

    nq = jnp.maximum(jnp.sqrt(jnp.sum(q * q, axis=1, keepdims=True)), 1e-12)
    nk2c = jnp.sum(k_next * k_next, axis=1, keepdims=True)
    nk = jnp.maximum(jnp.sqrt(nk2c), 1e-12)
    dqk = jnp.sum(q * k_next, axis=1, keepdims=True)
    cos = dqk / (nq * nk)

    pm = jnp.clip((1.0 - cos) * 0.5, 0.0, 1.0)
    prob = jnp.concatenate([jnp.ones((1, 1), f32), pm[:L - 1]], axis=0)

    mask = prob > 0.5
    a_col = jnp.where(mask, 1.0 - prob, 1.0)
    c_col = jnp.where(mask, prob, 0.0)
    a = jnp.broadcast_to(a_col, (L, T))
    c = jnp.broadcast_to(c_col, (L, T))

    t_io = jax.lax.broadcasted_iota(jnp.int32, (G, T, T), 1)
    j_io = jax.lax.broadcasted_iota(jnp.int32, (G, T, T), 2)
    Lm3 = jnp.where(t_io == j_io, 1.0, 0.0).astype(f32)
    a3 = a.reshape(G, T, T)
    s = 1
    while s < T:
        a_sh = jnp.concatenate([jnp.ones((G, s, T), f32), a3[:, :T - s]], axis=1)
        Lm_sh = jnp.concatenate([jnp.zeros((G, s, T), f32), Lm3[:, :T - s]], axis=1)
        Lm3 = Lm3 + a3 * Lm_sh
        a3 = a3 * a_sh
        s *= 2
    Lm = Lm3.reshape(L, T)
    a = a3.reshape(L, T)

    b_full = jnp.concatenate(
        [c * hs[:, j * T:(j + 1) * T] for j in range(D // T)], axis=1)

    s_locs = [
        jnp.dot(Lm[g * T:(g + 1) * T], b_full[g * T:(g + 1) * T],
                preferred_element_type=f32)
        for g in range(G)
    ]

    Sb = jnp.concatenate([sl[T - 1:T] for sl in s_locs], axis=0)
    Aa = jnp.concatenate(
        [a[g * T + T - 1:g * T + T] for g in range(G)], axis=0)
    s = 1
    while s < G:
        Sb_sh = jnp.concatenate([jnp.zeros((s, D), f32), Sb[:G - s]], axis=0)
        Aa_sh = jnp.concatenate([jnp.ones((s, T), f32), Aa[:G - s]], axis=0)
        Sb = Sb + jnp.concatenate(
            [Aa * Sb_sh[:, j * T:(j + 1) * T] for j in range(D // T)], axis=1)
        Aa = Aa * Aa_sh
        s *= 2
    S_prev = jnp.concatenate([jnp.zeros((1, D), f32), Sb[:G - 1]], axis=0)

    s_loc_full = jnp.concatenate(s_locs, axis=0)
    S_full = jnp.broadcast_to(S_prev.reshape(G, 1, D), (G, T, D)).reshape(L, D)
    corr = jnp.concatenate(
        [a * S_full[:, j * T:(j + 1) * T] for j in range(D // T)], axis=1)
    out_ref[0] = hs + s_loc_full + corr


def kernel(hidden_states, q_weight, k_weight):
    B, L, D = hidden_states.shape
    return pl.pallas_call(
        functools.partial(_hnet_kernel, L=L, D=D),
        grid=(B,),
        in_specs=[
            pl.BlockSpec((1, L, D), lambda b: (b, 0, 0)),
            pl.BlockSpec((D, D), lambda b: (0, 0)),
            pl.BlockSpec((D, D), lambda b: (0, 0)),
        ],
        out_specs=pl.BlockSpec((1, L, D), lambda b: (b, 0, 0)),
        out_shape=jax.ShapeDtypeStruct((B, L, D), hidden_states.dtype),
    )(hidden_states, q_weight, k_weight)


# speedup vs baseline: 12.9132x; 1.0147x over previous
import functools

import jax
import jax.numpy as jnp
from jax.experimental import pallas as pl

_T = 128


def _hnet_kernel(hs_ref, qw_ref, kw_ref, out_ref, *, L, D):
    T = _T
    G = L // T
    f32 = jnp.float32
    hs = hs_ref[0]
    qw = qw_ref[...]
    kw = kw_ref[...]

    dn = (((1,), (1,)), ((), ()))
    q = jax.lax.dot_general(hs, qw, dn, preferred_element_type=f32)
    k = jax.lax.dot_general(hs, kw, dn, preferred_element_type=f32)

    k_next = jnp.concatenate([k[1:], jnp.zeros((1, D), f32)], axis=0)

    nq = jnp.maximum(jnp.sqrt(jnp.sum(q * q, axis=1, keepdims=True)), 1e-12)
    nk2c = jnp.sum(k_next * k_next, axis=1, keepdims=True)
    nk = jnp.maximum(jnp.sqrt(nk2c), 1e-12)
    dqk = jnp.sum(q * k_next, axis=1, keepdims=True)
    cos = dqk / (nq * nk)

    pm = jnp.clip((1.0 - cos) * 0.5, 0.0, 1.0)
    prob = jnp.concatenate([jnp.ones((1, 1), f32), pm[:L - 1]], axis=0)

    prob128 = jnp.broadcast_to(prob, (L, T))
    mask = prob128 > 0.5
    a = jnp.where(mask, 1.0 - prob128, 1.0)
    c = jnp.where(mask, prob128, 0.0)

    t_io = jax.lax.broadcasted_iota(jnp.int32, (G, T, T), 1)
    j_io = jax.lax.broadcasted_iota(jnp.int32, (G, T, T), 2)
    a3 = a.reshape(G, T, T)
    Lm3 = (jnp.where(t_io == j_io, 1.0, 0.0)
           + jnp.where(t_io == j_io + 1, a3, 0.0)).astype(f32)
    a3 = a3 * jnp.concatenate([jnp.ones((G, 1, T), f32), a3[:, :T - 1]], axis=1)
    s = 2
    while s < T:
        a_sh = jnp.concatenate([jnp.ones((G, s, T), f32), a3[:, :T - s]], axis=1)
        Lm_sh = jnp.concatenate([jnp.zeros((G, s, T), f32), Lm3[:, :T - s]], axis=1)
        Lm3 = Lm3 + a3 * Lm_sh
        a3 = a3 * a_sh
        s *= 2
    Lm = Lm3.reshape(L, T)
    a = a3.reshape(L, T)

    b_full = jnp.concatenate(
        [c * hs[:, j * T:(j + 1) * T] for j in range(D // T)], axis=1)

    s_locs = [
        jnp.dot(Lm[g * T:(g + 1) * T], b_full[g * T:(g + 1) * T],
                preferred_element_type=f32)
        for g in range(G)
    ]

    Sb = jnp.concatenate([sl[T - 1:T] for sl in s_locs], axis=0)
    Aa = jnp.concatenate(
        [a[g * T + T - 1:g * T + T] for g in range(G)], axis=0)
    s = 1
    while s < G:
        Sb_sh = jnp.concatenate([jnp.zeros((s, D), f32), Sb[:G - s]], axis=0)
        Aa_sh = jnp.concatenate([jnp.ones((s, T), f32), Aa[:G - s]], axis=0)
        Sb = Sb + jnp.concatenate(
            [Aa * Sb_sh[:, j * T:(j + 1) * T] for j in range(D // T)], axis=1)
        Aa = Aa * Aa_sh
        s *= 2
    S_prev = jnp.concatenate([jnp.zeros((1, D), f32), Sb[:G - 1]], axis=0)

    s_loc_full = jnp.concatenate(s_locs, axis=0)
    S_full = jnp.broadcast_to(S_prev.reshape(G, 1, D), (G, T, D)).reshape(L, D)
    corr = jnp.concatenate(
        [a * S_full[:, j * T:(j + 1) * T] for j in range(D // T)], axis=1)
    out_ref[0] = hs + s_loc_full + corr


def kernel(hidden_states, q_weight, k_weight):
    B, L, D = hidden_states.shape
    return pl.pallas_call(
        functools.partial(_hnet_kernel, L=L, D=D),
        grid=(B,),
        in_specs=[
            pl.BlockSpec((1, L, D), lambda b: (b, 0, 0)),
            pl.BlockSpec((D, D), lambda b: (0, 0)),
            pl.BlockSpec((D, D), lambda b: (0, 0)),
        ],
        out_specs=pl.BlockSpec((1, L, D), lambda b: (b, 0, 0)),
        out_shape=jax.ShapeDtypeStruct((B, L, D), hidden_states.dtype),
    )(hidden_states, q_weight, k_weight)
